# baseline (speedup 1.0000x reference)
import functools

import jax
import jax.numpy as jnp
from jax import lax
from jax.experimental import pallas as pl
from jax.experimental.pallas import tpu as pltpu
from jax.experimental.pallas import tpu_sc as plsc

NC = 2
NS = 16
NW = NC * NS
L = 16

N_POS = 1024
NEG_TOT = 20480
D = 128
POSW = N_POS // NW
NEGW = NEG_TOT // NW
NEG_CHUNKS = NEGW // 128
CAP = 4096
WFW = 144
FSTRIDE = 128
WSTEPS = 9


def _sc_main_body(nnz, nf, fsteps,
                  features_h, movies_h, ratings_h, pos_idx_h, bu_h,
                  bi_h, users2d_h, fence_h,
                  wfd_h, bug_h, bip_h, fpos_h,
                  pidx_v, u_v, m_v, urow_v, ubuf, fence_v, winidx_v, winbuf,
                  curs_v,
                  flat_idx, flat_pid, flat_val, mbig, rbig, rsc_v,
                  rowbuf0, rowbuf1, rowbuf2, rowbuf3,
                  idxstg0, idxstg1, idxstg2, idxstg3, wf_v,
                  bu_b, bi_b, fpos_b, sem, semA, semR0, semR1, semR2, semR3,
                  sem2, semF):
    wid = lax.axis_index("s") * NC + lax.axis_index("c")
    base_p = wid * POSW

    iota16 = lax.iota(jnp.int32, L)

    cf = pltpu.async_copy(fence_h, fence_v, semF)

    pltpu.sync_copy(pos_idx_h.at[pl.ds(base_p, POSW)], pidx_v)
    p0 = pidx_v[pl.ds(0, L)]
    p1 = pidx_v[pl.ds(L, L)]
    urow_v[pl.ds(0, L)] = lax.shift_right_logical(p0, 7)
    urow_v[pl.ds(L, L)] = lax.shift_right_logical(p1, 7)
    cu = pltpu.async_copy(users2d_h.at[urow_v], ubuf, sem)
    cm = pltpu.async_copy(movies_h.at[pidx_v], m_v, sem)
    cu.wait()
    cm.wait()
    u0 = plsc.load_gather(ubuf, [iota16, p0 & (FSTRIDE - 1)])
    u1 = plsc.load_gather(ubuf, [L + iota16, p1 & (FSTRIDE - 1)])
    u_v[pl.ds(0, L)] = u0
    u_v[pl.ds(L, L)] = u1

    pend = [pltpu.async_copy(bu_h.at[u_v], bu_b, sem2),
            pltpu.async_copy(bi_h.at[m_v], bi_b, sem2),
            pltpu.async_copy(features_h.at[m_v], fpos_b, sem2)]

    sscope = jax.named_scope("ph_search")
    sscope.__enter__()
    cf.wait()
    uu = [u0, u1, u0, u1]
    lo_ = [jnp.zeros((L,), jnp.int32) for _ in range(4)]
    hi_ = [jnp.full((L,), nf, jnp.int32) for _ in range(4)]

    for _step in range(fsteps):
        for i in range(4):
            mid = (lo_[i] + hi_[i]) >> 1
            pv = plsc.load_gather(fence_v, [jnp.minimum(mid, nf - 1)])
            cond = (pv >= uu[i]) if i < 2 else (pv > uu[i])
            hi_[i] = jnp.where(cond, mid, hi_[i])
            lo_[i] = jnp.where(cond, lo_[i], mid + 1)

    r_ = [jnp.clip(lo_[i] - 1, 0, nf - 2) for i in range(4)]
    for i in range(4):
        winidx_v[pl.ds(i * L, L)] = r_[i]
        winidx_v[pl.ds(4 * L + i * L, L)] = r_[i] + 1
    pltpu.async_copy(users2d_h.at[winidx_v], winbuf, semF).wait()

    bounds = []
    for i in range(4):
        lo = jnp.zeros((L,), jnp.int32)
        hi = jnp.full((L,), 2 * FSTRIDE, jnp.int32)
        qrow = i * L + iota16
        for _step in range(WSTEPS):
            mid = (lo + hi) >> 1
            midc = jnp.minimum(mid, 2 * FSTRIDE - 1)
            row = qrow + (midc >> 7) * 64
            col = midc & (FSTRIDE - 1)
            pv = plsc.load_gather(winbuf, [row, col])
            cond = (pv >= uu[i]) if i < 2 else (pv > uu[i])
            hi = jnp.where(cond, mid, hi)
            lo = jnp.where(cond, lo, mid + 1)
        bounds.append(r_[i] * FSTRIDE + lo)

    starts = bounds[0:2]
    ends = bounds[2:4]
    sscope.__exit__(None, None, None)

    zero16 = jnp.zeros((L,), jnp.float32)

    def _zbody(i, _):
        wf_v[pl.ds(i * L, L)] = zero16
        return 0

    lax.fori_loop(0, (POSW + 1) * WFW // L, _zbody, 0)

    curs_v[pl.ds(0, L)] = starts[0]
    curs_v[pl.ds(L, L)] = starts[1]

    def _more(c0, c1):
        n0 = plsc.all_reduce_population_count(c0 < ends[0])
        n1 = plsc.all_reduce_population_count(c1 < ends[1])
        return n0[0] + n1[0]

    def _pass_body(_unused):
      with jax.named_scope("ph_build"):
        curs = [curs_v[pl.ds(0, L)], curs_v[pl.ds(L, L)]]
        nfill = jnp.int32(0)
        for j in range(POSW):
            c = curs[j // L][j % L]
            e = ends[j // L][j % L]
            take = jnp.maximum(jnp.minimum(e - c, CAP - nfill), 0)

            def _wbody(carry2, c=c, take=take, nfill=nfill, j=j):
                k, _ = carry2
                valid = (k + iota16) < take
                idxv = jnp.minimum(c + k + iota16, nnz - 1)
                flat_idx[pl.ds(nfill + k, L)] = idxv
                flat_pid[pl.ds(nfill + k, L)] = jnp.full((L,), j, jnp.int32)
                flat_val[pl.ds(nfill + k, L)] = jnp.where(valid, 1.0, 0.0)
                return k + L, 0

            lax.while_loop(lambda c2, take=take: c2[0] < take, _wbody,
                           (jnp.int32(0), 0))
            curs[j // L] = jnp.where(iota16 == (j % L), c + take,
                                     curs[j // L])
            nfill = nfill + ((take + L - 1) // L) * L

        def _padbody(carry2):
            k, _ = carry2
            flat_idx[pl.ds(k, L)] = jnp.zeros((L,), jnp.int32)
            flat_pid[pl.ds(k, L)] = jnp.full((L,), POSW, jnp.int32)
            flat_val[pl.ds(k, L)] = jnp.zeros((L,), jnp.float32)
            return k + L, 0

        nfill, _ = lax.while_loop(lambda c2: (c2[0] % 128) != 0, _padbody,
                                  (nfill, 0))
        nchunks = nfill // 128

      with jax.named_scope("ph_mr"):
        def _fire(ci, _):
            sl = flat_idx.at[pl.ds(ci * 128, 128)]
            pltpu.async_copy(movies_h.at[sl], mbig.at[pl.ds(ci * 128, 128)],
                             semA)
            pltpu.async_copy(ratings_h.at[sl], rbig.at[pl.ds(ci * 128, 128)],
                             semA)
            return 0

        lax.fori_loop(0, nchunks, _fire, 0)

        def _drainA(ci, _):
            sl = flat_idx.at[pl.ds(0, 128)]
            pltpu.make_async_copy(movies_h.at[sl],
                                  mbig.at[pl.ds(0, 128)], semA).wait()
            pltpu.make_async_copy(ratings_h.at[sl],
                                  rbig.at[pl.ds(0, 128)], semA).wait()
            return 0

        lax.fori_loop(0, nchunks, _drainA, 0)

      with jax.named_scope("ph_rows"):
        rowbufs = [rowbuf0, rowbuf1, rowbuf2, rowbuf3]
        semRs = [semR0, semR1, semR2, semR3]
        idxstgs = [idxstg0, idxstg1, idxstg2, idxstg3]
        depth = len(rowbufs)

        def _fire_rows(ci, b):
            def _cp(l2, _):
                idxstgs[b][pl.ds(l2 * L, L)] = mbig[pl.ds(ci * 128 + l2 * L,
                                                          L)]
                return 0

            lax.fori_loop(0, 128 // L, _cp, 0)
            pltpu.make_async_copy(features_h.at[idxstgs[b]], rowbufs[b],
                                  semRs[b]).start()

        def _rows_copy(ci, b):
            return pltpu.make_async_copy(
                features_h.at[idxstgs[b]], rowbufs[b], semRs[b])

        for pb in range(4):
            @pl.when(nchunks > pb)
            def _prime(pb=pb):
                _fire_rows(pb, pb)

        def _g(g, _):
            for b in range(4):
                ci = g * 4 + b

                @pl.when(ci < nchunks)
                def _do(ci=ci, b=b):
                    _rows_copy(ci, b).wait()
                    off = ci * 128
                    rb = rowbufs[b]

                    def _prescale(l2, _2):
                        goff = off + l2 * L
                        rsc_v[pl.ds(l2 * L, L)] = (rbig[pl.ds(goff, L)] *
                                                   flat_val[pl.ds(goff, L)])
                        return 0

                    lax.fori_loop(0, 128 // L, _prescale, 0)

                    def _edges(l2, _2):
                        pid16 = flat_pid[pl.ds(off + l2 * L, L)]
                        pid = pid16[0]
                        base = pid * WFW
                        r16 = rsc_v[pl.ds(l2 * L, L)]
                        acc = [jnp.zeros((L,), jnp.float32)
                               for _ in range(D // L)]
                        for t in range(L):
                            rv = plsc.load_gather(
                                rsc_v, [jnp.full((L,), l2 * L + t,
                                                 jnp.int32)])
                            for d in range(D // L):
                                acc[d] = acc[d] + (
                                    rb[l2 * L + t, pl.ds(d * L, L)] * rv)
                        for d in range(D // L):
                            plsc.addupdate(wf_v.at[pl.ds(base + d * L, L)],
                                           acc[d])
                        plsc.addupdate(wf_v.at[pl.ds(base + D, L)], r16)
                        return 0

                    lax.fori_loop(0, 128 // L, _edges, 0)

                    @pl.when(ci + depth < nchunks)
                    def _next():
                        _fire_rows(ci + depth, b)
            return 0

        lax.fori_loop(0, (nchunks + 3) // 4, _g, 0)

        curs_v[pl.ds(0, L)] = curs[0]
        curs_v[pl.ds(L, L)] = curs[1]
        return _more(curs[0], curs[1])

    lax.while_loop(lambda m: m > 0, _pass_body,
                   _more(starts[0], starts[1]))

    oscope = jax.named_scope("ph_out")
    oscope.__enter__()
    for c in pend:
        c.wait()
    pltpu.sync_copy(bu_b, bug_h.at[pl.ds(base_p, POSW)])
    pltpu.sync_copy(bi_b, bip_h.at[pl.ds(base_p, POSW)])
    pltpu.sync_copy(fpos_b, fpos_h.at[pl.ds(base_p, POSW)])
    pltpu.sync_copy(wf_v.at[pl.ds(0, POSW * WFW)],
                    wfd_h.at[pl.ds(base_p * WFW, POSW * WFW)])
    oscope.__exit__(None, None, None)


def _sc_neg_body(features_h, neg_idx_h, bi_h,
                 fneg_h, bin_h,
                 nidx_v, fneg_b, bin_b, sem):
    wid = lax.axis_index("s") * NC + lax.axis_index("c")
    base_n = wid * NEGW
    pltpu.sync_copy(neg_idx_h.at[pl.ds(base_n, NEGW)], nidx_v)
    pend = []
    for _rep in range(3):
        for k in range(NEG_CHUNKS):
            sl = nidx_v.at[pl.ds(k * 128, 128)]
            pend.append(pltpu.async_copy(features_h.at[sl],
                                         fneg_b.at[pl.ds(k * 128, 128)],
                                         sem))
    for k in range(NEG_CHUNKS):
        sl = nidx_v.at[pl.ds(k * 128, 128)]
        pend.append(pltpu.async_copy(bi_h.at[sl],
                                     bin_b.at[pl.ds(k * 128, 128)], sem))
    for c in pend:
        c.wait()
    pltpu.sync_copy(fneg_b, fneg_h.at[pl.ds(base_n, NEGW)])
    pltpu.sync_copy(bin_b, bin_h.at[pl.ds(base_n, NEGW)])


def _tc_body(wfd_ref, fpos_ref, fneg_ref, bug_ref, bip_ref, bin_ref,
             wp_ref, bp_ref, wq_ref, bq_ref, out_ref):
    nsamp = NEG_TOT // N_POS
    wfd = wfd_ref[:]
    wf = wfd[:, :D]
    deg = jnp.sum(wfd[:, D:D + 16], axis=1)
    wp = wp_ref[:]
    wq = wq_ref[:]
    bp = bp_ref[:]
    bq = bq_ref[:]
    dn = (((1,), (1,)), ((), ()))
    acc_p = lax.dot_general(wf, wp, dn) + deg[:, None] * bp[None, :]
    f_pos = fpos_ref[:]
    p_pos = lax.dot_general(f_pos, wp, dn) + bp[None, :]
    q_pos = lax.dot_general(f_pos, wq, dn) + bq[None, :]
    f_neg = fneg_ref[:]
    p_neg = lax.dot_general(f_neg, wp, dn) + bp[None, :]
    q_neg = lax.dot_general(f_neg, wq, dn) + bq[None, :]

    denom = jnp.maximum(deg - 1.0, 1.0)
    bu = bug_ref[:]
    r_pos = (bu + bip_ref[:] +
             jnp.sum((acc_p - p_pos) * q_pos, axis=1) / denom)

    q_neg3 = q_neg.reshape(N_POS, nsamp, 16)
    t1 = jnp.sum(acc_p[:, None, :] * q_neg3, axis=2)
    t2 = jnp.sum(p_neg * q_neg, axis=1).reshape(N_POS, nsamp)
    r_neg = bu[:, None] + bin_ref[:] + (t1 - t2) / denom[:, None]

    diff = 1.0 - (r_pos[:, None] - r_neg)
    out_ref[...] = jnp.reshape(jnp.sum(diff * diff) * 0.5, (1, 1))


def kernel(features, users, movies, ratings, pos_idx, neg_item_idx,
           neg_sample_size, W_p, b_p, W_q, b_q, b_u, b_i):
    nnz = users.shape[0]
    nf = nnz // FSTRIDE
    fsteps = max(1, (nf + 1 - 1).bit_length())
    users2d = users.reshape(nf, FSTRIDE)
    fence = users2d[:, 0]
    mesh = plsc.VectorSubcoreMesh(core_axis_name="c", subcore_axis_name="s",
                                  num_cores=NC, num_subcores=NS)
    f32 = jnp.float32
    i32 = jnp.int32
    sc_main = pl.kernel(
        functools.partial(_sc_main_body, nnz, nf, fsteps),
        out_type=[
            jax.ShapeDtypeStruct((N_POS * WFW,), f32),
            jax.ShapeDtypeStruct((N_POS,), f32),
            jax.ShapeDtypeStruct((N_POS,), f32),
            jax.ShapeDtypeStruct((N_POS, D), f32),
        ],
        mesh=mesh,
        scratch_types=[
            pltpu.VMEM((POSW,), i32),
            pltpu.VMEM((POSW,), i32),
            pltpu.VMEM((POSW,), i32),
            pltpu.VMEM((POSW,), i32),
            pltpu.VMEM((POSW, FSTRIDE), i32),
            pltpu.VMEM((nf,), i32),
            pltpu.VMEM((8 * L,), i32),
            pltpu.VMEM((8 * L, FSTRIDE), i32),
            pltpu.VMEM((2 * L,), i32),
            pltpu.VMEM((CAP,), i32),
            pltpu.VMEM((CAP,), i32),
            pltpu.VMEM((CAP,), f32),
            pltpu.VMEM((CAP,), i32),
            pltpu.VMEM((CAP,), f32),
            pltpu.VMEM((128,), f32),
            pltpu.VMEM((128, D), f32),
            pltpu.VMEM((128, D), f32),
            pltpu.VMEM((128, D), f32),
            pltpu.VMEM((128, D), f32),
            pltpu.VMEM((128,), i32),
            pltpu.VMEM((128,), i32),
            pltpu.VMEM((128,), i32),
            pltpu.VMEM((128,), i32),
            pltpu.VMEM(((POSW + 1) * WFW,), f32),
            pltpu.VMEM((POSW,), f32),
            pltpu.VMEM((POSW,), f32),
            pltpu.VMEM((POSW, D), f32),
            pltpu.SemaphoreType.DMA,
            pltpu.SemaphoreType.DMA,
            pltpu.SemaphoreType.DMA,
            pltpu.SemaphoreType.DMA,
            pltpu.SemaphoreType.DMA,
            pltpu.SemaphoreType.DMA,
            pltpu.SemaphoreType.DMA,
            pltpu.SemaphoreType.DMA,
        ],
        compiler_params=pltpu.CompilerParams(needs_layout_passes=False),
    )
    wfd, bug, bip, fpos = sc_main(features, movies, ratings, pos_idx,
                                  b_u, b_i, users2d, fence)

    sc_neg = pl.kernel(
        _sc_neg_body,
        out_type=[
            jax.ShapeDtypeStruct((NEG_TOT, D), f32),
            jax.ShapeDtypeStruct((NEG_TOT,), f32),
        ],
        mesh=mesh,
        scratch_types=[
            pltpu.VMEM((NEGW,), i32),
            pltpu.VMEM((NEGW, D), f32),
            pltpu.VMEM((NEGW,), f32),
            pltpu.SemaphoreType.DMA,
        ],
        compiler_params=pltpu.CompilerParams(needs_layout_passes=False),
    )
    fneg, bin_ = sc_neg(features, neg_item_idx, b_i)

    wfd2 = wfd.reshape(N_POS, WFW)
    bin2 = bin_.reshape(N_POS, NEG_TOT // N_POS)
    loss = pl.pallas_call(
        _tc_body,
        out_shape=jax.ShapeDtypeStruct((1, 1), f32),
    )(wfd2, fpos, fneg, bug, bip, bin2, W_p, b_p, W_q, b_q)
    return loss[0, 0]


# speedup vs baseline: 1.6449x; 1.6449x over previous
import functools

import jax
import jax.numpy as jnp
from jax import lax
from jax.experimental import pallas as pl
from jax.experimental.pallas import tpu as pltpu
from jax.experimental.pallas import tpu_sc as plsc

NC = 2
NS = 16
NW = NC * NS
L = 16

N_POS = 1024
NEG_TOT = 20480
D = 128
POSW = N_POS // NW
NEGW = NEG_TOT // NW
NEG_CHUNKS = NEGW // 128
CAP = 4096
WFW = 144
FSTRIDE = 128
WSTEPS = 9


def _sc_main_body(nnz, nf, fsteps,
                  features_h, movies_h, ratings_h, pos_idx_h, bu_h,
                  bi_h, users2d_h, fence_h,
                  wfd_h, bug_h, bip_h, fpos_h,
                  pidx_v, u_v, m_v, urow_v, ubuf, fence_v, winidx_v, winbuf,
                  curs_v,
                  flat_idx, flat_pid, flat_val, mbig, rbig, rsc_v,
                  rowbuf0, rowbuf1, rowbuf2, rowbuf3,
                  idxstg0, idxstg1, idxstg2, idxstg3, wf_v,
                  bu_b, bi_b, fpos_b, sem, semA, semR0, semR1, semR2, semR3,
                  sem2, semF):
    wid = lax.axis_index("s") * NC + lax.axis_index("c")
    base_p = wid * POSW

    iota16 = lax.iota(jnp.int32, L)

    cf = pltpu.async_copy(fence_h, fence_v, semF)

    pltpu.sync_copy(pos_idx_h.at[pl.ds(base_p, POSW)], pidx_v)
    p0 = pidx_v[pl.ds(0, L)]
    p1 = pidx_v[pl.ds(L, L)]
    urow_v[pl.ds(0, L)] = lax.shift_right_logical(p0, 7)
    urow_v[pl.ds(L, L)] = lax.shift_right_logical(p1, 7)
    cu = pltpu.async_copy(users2d_h.at[urow_v], ubuf, sem)
    cm = pltpu.async_copy(movies_h.at[pidx_v], m_v, sem)
    cu.wait()
    cm.wait()
    u0 = plsc.load_gather(ubuf, [iota16, p0 & (FSTRIDE - 1)])
    u1 = plsc.load_gather(ubuf, [L + iota16, p1 & (FSTRIDE - 1)])
    u_v[pl.ds(0, L)] = u0
    u_v[pl.ds(L, L)] = u1

    pend = [pltpu.async_copy(bu_h.at[u_v], bu_b, sem2),
            pltpu.async_copy(bi_h.at[m_v], bi_b, sem2),
            pltpu.async_copy(features_h.at[m_v], fpos_b, sem2)]

    sscope = jax.named_scope("ph_search")
    sscope.__enter__()
    cf.wait()
    uu = [u0, u1, u0, u1]
    lo_ = [jnp.zeros((L,), jnp.int32) for _ in range(4)]
    hi_ = [jnp.full((L,), nf, jnp.int32) for _ in range(4)]

    for _step in range(fsteps):
        for i in range(4):
            mid = (lo_[i] + hi_[i]) >> 1
            pv = plsc.load_gather(fence_v, [jnp.minimum(mid, nf - 1)])
            cond = (pv >= uu[i]) if i < 2 else (pv > uu[i])
            hi_[i] = jnp.where(cond, mid, hi_[i])
            lo_[i] = jnp.where(cond, lo_[i], mid + 1)

    r_ = [jnp.clip(lo_[i] - 1, 0, nf - 2) for i in range(4)]
    for i in range(4):
        winidx_v[pl.ds(i * L, L)] = r_[i]
        winidx_v[pl.ds(4 * L + i * L, L)] = r_[i] + 1
    pltpu.async_copy(users2d_h.at[winidx_v], winbuf, semF).wait()

    bounds = []
    for i in range(4):
        lo = jnp.zeros((L,), jnp.int32)
        hi = jnp.full((L,), 2 * FSTRIDE, jnp.int32)
        qrow = i * L + iota16
        for _step in range(WSTEPS):
            mid = (lo + hi) >> 1
            midc = jnp.minimum(mid, 2 * FSTRIDE - 1)
            row = qrow + (midc >> 7) * 64
            col = midc & (FSTRIDE - 1)
            pv = plsc.load_gather(winbuf, [row, col])
            cond = (pv >= uu[i]) if i < 2 else (pv > uu[i])
            hi = jnp.where(cond, mid, hi)
            lo = jnp.where(cond, lo, mid + 1)
        bounds.append(r_[i] * FSTRIDE + lo)

    starts = bounds[0:2]
    ends = bounds[2:4]
    sscope.__exit__(None, None, None)

    zero16 = jnp.zeros((L,), jnp.float32)

    def _zbody(i, _):
        wf_v[pl.ds(i * L, L)] = zero16
        return 0

    lax.fori_loop(0, (POSW + 1) * WFW // L, _zbody, 0)

    curs_v[pl.ds(0, L)] = starts[0]
    curs_v[pl.ds(L, L)] = starts[1]

    def _more(c0, c1):
        n0 = plsc.all_reduce_population_count(c0 < ends[0])
        n1 = plsc.all_reduce_population_count(c1 < ends[1])
        return n0[0] + n1[0]

    def _pass_body(_unused):
      with jax.named_scope("ph_build"):
        curs = [curs_v[pl.ds(0, L)], curs_v[pl.ds(L, L)]]
        nfill = jnp.int32(0)
        for j in range(POSW):
            c = curs[j // L][j % L]
            e = ends[j // L][j % L]
            take = jnp.maximum(jnp.minimum(e - c, CAP - nfill), 0)

            def _wbody(carry2, c=c, take=take, nfill=nfill, j=j):
                k, _ = carry2
                valid = (k + iota16) < take
                idxv = jnp.minimum(c + k + iota16, nnz - 1)
                flat_idx[pl.ds(nfill + k, L)] = idxv
                flat_pid[pl.ds(nfill + k, L)] = jnp.full((L,), j, jnp.int32)
                flat_val[pl.ds(nfill + k, L)] = jnp.where(valid, 1.0, 0.0)
                return k + L, 0

            lax.while_loop(lambda c2, take=take: c2[0] < take, _wbody,
                           (jnp.int32(0), 0))
            curs[j // L] = jnp.where(iota16 == (j % L), c + take,
                                     curs[j // L])
            nfill = nfill + ((take + L - 1) // L) * L

        def _padbody(carry2):
            k, _ = carry2
            flat_idx[pl.ds(k, L)] = (k & 1023) + iota16
            flat_pid[pl.ds(k, L)] = jnp.full((L,), POSW, jnp.int32)
            flat_val[pl.ds(k, L)] = jnp.zeros((L,), jnp.float32)
            return k + L, 0

        nfill, _ = lax.while_loop(lambda c2: (c2[0] % 512) != 0, _padbody,
                                  (nfill, 0))
        nchunks = nfill // 128

      with jax.named_scope("ph_mr"):
        def _fire(ci, _):
            sl = flat_idx.at[pl.ds(ci * 128, 128)]
            pltpu.async_copy(movies_h.at[sl], mbig.at[pl.ds(ci * 128, 128)],
                             semA)
            pltpu.async_copy(ratings_h.at[sl], rbig.at[pl.ds(ci * 128, 128)],
                             semA)
            return 0

        lax.fori_loop(0, nchunks, _fire, 0)

        def _drainA(ci, _):
            sl = flat_idx.at[pl.ds(0, 128)]
            pltpu.make_async_copy(movies_h.at[sl],
                                  mbig.at[pl.ds(0, 128)], semA).wait()
            pltpu.make_async_copy(ratings_h.at[sl],
                                  rbig.at[pl.ds(0, 128)], semA).wait()
            return 0

        lax.fori_loop(0, nchunks, _drainA, 0)

      with jax.named_scope("ph_rows"):
        rowbufs = [rowbuf0, rowbuf1, rowbuf2, rowbuf3]
        semRs = [semR0, semR1, semR2, semR3]
        idxstgs = [idxstg0, idxstg1, idxstg2, idxstg3]
        depth = len(rowbufs)

        def _fire_rows(ci, b):
            def _cp(l2, _):
                idxstgs[b][pl.ds(l2 * L, L)] = mbig[pl.ds(ci * 128 + l2 * L,
                                                          L)]
                return 0

            lax.fori_loop(0, 128 // L, _cp, 0)
            pltpu.make_async_copy(features_h.at[idxstgs[b]], rowbufs[b],
                                  semRs[b]).start()

        def _rows_copy(ci, b):
            return pltpu.make_async_copy(
                features_h.at[idxstgs[b]], rowbufs[b], semRs[b])

        def _g(g, _):
            for b in range(4):
                _fire_rows(g * 4 + b, b)
            for b in range(4):
                ci = g * 4 + b
                if True:
                    _rows_copy(ci, b).wait()
                    off = ci * 128
                    rb = rowbufs[b]

                    def _prescale(l2, _2):
                        goff = off + l2 * L
                        rsc_v[pl.ds(l2 * L, L)] = (rbig[pl.ds(goff, L)] *
                                                   flat_val[pl.ds(goff, L)])
                        return 0

                    lax.fori_loop(0, 128 // L, _prescale, 0)

                    def _edges(l2, _2):
                        pid16 = flat_pid[pl.ds(off + l2 * L, L)]
                        pid = pid16[0]
                        base = pid * WFW
                        r16 = rsc_v[pl.ds(l2 * L, L)]
                        acc = [jnp.zeros((L,), jnp.float32)
                               for _ in range(D // L)]
                        for t in range(L):
                            rv = plsc.load_gather(
                                rsc_v, [jnp.full((L,), l2 * L + t,
                                                 jnp.int32)])
                            for d in range(D // L):
                                acc[d] = acc[d] + (
                                    rb[l2 * L + t, pl.ds(d * L, L)] * rv)
                        for d in range(D // L):
                            plsc.addupdate(wf_v.at[pl.ds(base + d * L, L)],
                                           acc[d])
                        plsc.addupdate(wf_v.at[pl.ds(base + D, L)], r16)
                        return 0

                    lax.fori_loop(0, 128 // L, _edges, 0)
            return 0

        lax.fori_loop(0, nchunks // 4, _g, 0)

        curs_v[pl.ds(0, L)] = curs[0]
        curs_v[pl.ds(L, L)] = curs[1]
        return _more(curs[0], curs[1])

    lax.while_loop(lambda m: m > 0, _pass_body,
                   _more(starts[0], starts[1]))

    oscope = jax.named_scope("ph_out")
    oscope.__enter__()
    for c in pend:
        c.wait()
    pltpu.sync_copy(bu_b, bug_h.at[pl.ds(base_p, POSW)])
    pltpu.sync_copy(bi_b, bip_h.at[pl.ds(base_p, POSW)])
    pltpu.sync_copy(fpos_b, fpos_h.at[pl.ds(base_p, POSW)])
    pltpu.sync_copy(wf_v.at[pl.ds(0, POSW * WFW)],
                    wfd_h.at[pl.ds(base_p * WFW, POSW * WFW)])
    oscope.__exit__(None, None, None)


def _sc_neg_body(features_h, neg_idx_h, bi_h,
                 fneg_h, bin_h,
                 nidx_v, fneg_b, bin_b, sem):
    wid = lax.axis_index("s") * NC + lax.axis_index("c")
    base_n = wid * NEGW
    pltpu.sync_copy(neg_idx_h.at[pl.ds(base_n, NEGW)], nidx_v)
    pend = []
    for k in range(NEG_CHUNKS):
        sl = nidx_v.at[pl.ds(k * 128, 128)]
        pend.append(pltpu.async_copy(features_h.at[sl],
                                     fneg_b.at[pl.ds(k * 128, 128)], sem))
        pend.append(pltpu.async_copy(bi_h.at[sl],
                                     bin_b.at[pl.ds(k * 128, 128)], sem))
    for c in pend:
        c.wait()
    pltpu.sync_copy(fneg_b, fneg_h.at[pl.ds(base_n, NEGW)])
    pltpu.sync_copy(bin_b, bin_h.at[pl.ds(base_n, NEGW)])


def _tc_body(wfd_ref, fpos_ref, fneg_ref, bug_ref, bip_ref, bin_ref,
             wp_ref, bp_ref, wq_ref, bq_ref, out_ref):
    nsamp = NEG_TOT // N_POS
    wfd = wfd_ref[:]
    wf = wfd[:, :D]
    deg = jnp.sum(wfd[:, D:D + 16], axis=1)
    wp = wp_ref[:]
    wq = wq_ref[:]
    bp = bp_ref[:]
    bq = bq_ref[:]
    dn = (((1,), (1,)), ((), ()))
    acc_p = lax.dot_general(wf, wp, dn) + deg[:, None] * bp[None, :]
    f_pos = fpos_ref[:]
    p_pos = lax.dot_general(f_pos, wp, dn) + bp[None, :]
    q_pos = lax.dot_general(f_pos, wq, dn) + bq[None, :]
    f_neg = fneg_ref[:]
    p_neg = lax.dot_general(f_neg, wp, dn) + bp[None, :]
    q_neg = lax.dot_general(f_neg, wq, dn) + bq[None, :]

    denom = jnp.maximum(deg - 1.0, 1.0)
    bu = bug_ref[:]
    r_pos = (bu + bip_ref[:] +
             jnp.sum((acc_p - p_pos) * q_pos, axis=1) / denom)

    q_neg3 = q_neg.reshape(N_POS, nsamp, 16)
    t1 = jnp.sum(acc_p[:, None, :] * q_neg3, axis=2)
    t2 = jnp.sum(p_neg * q_neg, axis=1).reshape(N_POS, nsamp)
    r_neg = bu[:, None] + bin_ref[:] + (t1 - t2) / denom[:, None]

    diff = 1.0 - (r_pos[:, None] - r_neg)
    out_ref[...] = jnp.reshape(jnp.sum(diff * diff) * 0.5, (1, 1))


def kernel(features, users, movies, ratings, pos_idx, neg_item_idx,
           neg_sample_size, W_p, b_p, W_q, b_q, b_u, b_i):
    nnz = users.shape[0]
    nf = nnz // FSTRIDE
    fsteps = max(1, (nf + 1 - 1).bit_length())
    users2d = users.reshape(nf, FSTRIDE)
    fence = users2d[:, 0]
    mesh = plsc.VectorSubcoreMesh(core_axis_name="c", subcore_axis_name="s",
                                  num_cores=NC, num_subcores=NS)
    f32 = jnp.float32
    i32 = jnp.int32
    sc_main = pl.kernel(
        functools.partial(_sc_main_body, nnz, nf, fsteps),
        out_type=[
            jax.ShapeDtypeStruct((N_POS * WFW,), f32),
            jax.ShapeDtypeStruct((N_POS,), f32),
            jax.ShapeDtypeStruct((N_POS,), f32),
            jax.ShapeDtypeStruct((N_POS, D), f32),
        ],
        mesh=mesh,
        scratch_types=[
            pltpu.VMEM((POSW,), i32),
            pltpu.VMEM((POSW,), i32),
            pltpu.VMEM((POSW,), i32),
            pltpu.VMEM((POSW,), i32),
            pltpu.VMEM((POSW, FSTRIDE), i32),
            pltpu.VMEM((nf,), i32),
            pltpu.VMEM((8 * L,), i32),
            pltpu.VMEM((8 * L, FSTRIDE), i32),
            pltpu.VMEM((2 * L,), i32),
            pltpu.VMEM((CAP,), i32),
            pltpu.VMEM((CAP,), i32),
            pltpu.VMEM((CAP,), f32),
            pltpu.VMEM((CAP,), i32),
            pltpu.VMEM((CAP,), f32),
            pltpu.VMEM((128,), f32),
            pltpu.VMEM((128, D), f32),
            pltpu.VMEM((128, D), f32),
            pltpu.VMEM((128, D), f32),
            pltpu.VMEM((128, D), f32),
            pltpu.VMEM((128,), i32),
            pltpu.VMEM((128,), i32),
            pltpu.VMEM((128,), i32),
            pltpu.VMEM((128,), i32),
            pltpu.VMEM(((POSW + 1) * WFW,), f32),
            pltpu.VMEM((POSW,), f32),
            pltpu.VMEM((POSW,), f32),
            pltpu.VMEM((POSW, D), f32),
            pltpu.SemaphoreType.DMA,
            pltpu.SemaphoreType.DMA,
            pltpu.SemaphoreType.DMA,
            pltpu.SemaphoreType.DMA,
            pltpu.SemaphoreType.DMA,
            pltpu.SemaphoreType.DMA,
            pltpu.SemaphoreType.DMA,
            pltpu.SemaphoreType.DMA,
        ],
        compiler_params=pltpu.CompilerParams(needs_layout_passes=False),
    )
    wfd, bug, bip, fpos = sc_main(features, movies, ratings, pos_idx,
                                  b_u, b_i, users2d, fence)

    sc_neg = pl.kernel(
        _sc_neg_body,
        out_type=[
            jax.ShapeDtypeStruct((NEG_TOT, D), f32),
            jax.ShapeDtypeStruct((NEG_TOT,), f32),
        ],
        mesh=mesh,
        scratch_types=[
            pltpu.VMEM((NEGW,), i32),
            pltpu.VMEM((NEGW, D), f32),
            pltpu.VMEM((NEGW,), f32),
            pltpu.SemaphoreType.DMA,
        ],
        compiler_params=pltpu.CompilerParams(needs_layout_passes=False),
    )
    fneg, bin_ = sc_neg(features, neg_item_idx, b_i)

    wfd2 = wfd.reshape(N_POS, WFW)
    bin2 = bin_.reshape(N_POS, NEG_TOT // N_POS)
    loss = pl.pallas_call(
        _tc_body,
        out_shape=jax.ShapeDtypeStruct((1, 1), f32),
    )(wfd2, fpos, fneg, bug, bip, bin2, W_p, b_p, W_q, b_q)
    return loss[0, 0]
